# Initial kernel scaffold; baseline (speedup 1.0000x reference)
import functools

import jax
import jax.numpy as jnp
from jax import lax
from jax.experimental import pallas as pl
from jax.experimental.pallas import tpu as pltpu
from jax.experimental.pallas import tpu_sc as plsc

N = 10000
E = 320000
D = 128
HALF = 64
NC = 2
NS = 16

CHUNK = 128
NCHUNKS = 157
PER_TILE = CHUNK * NCHUNKS
EPAD = NS * PER_TILE
NPAD = 10016
ZROWS = NPAD // NS
OROWS = N // NS


def _sc_body(with_cnt, xcat, src4, dst3, u3, zg, zc, *rest):
    if with_cnt:
        g_out, h_out, cnt_out, accg, acch, acccnt, srcv, dstv, uv, rows, hrows, onesv = rest
    else:
        g_out, h_out, accg, acch, acccnt, srcv, dstv, uv, rows, hrows, onesv = rest
        cnt_out = None

    c = lax.axis_index("c")
    s = lax.axis_index("s")

    pltpu.sync_copy(zg, accg.at[pl.ds(s * ZROWS, ZROWS)])
    pltpu.sync_copy(zg, acch.at[pl.ds(s * ZROWS, ZROWS)])
    if with_cnt:
        pltpu.sync_copy(zc, acccnt.at[pl.ds(s * ZROWS, ZROWS)])

    pltpu.sync_copy(src4.at[c, s], srcv)
    pltpu.sync_copy(dst3.at[s], dstv)
    pltpu.sync_copy(u3.at[s], uv)

    if with_cnt:
        def ones_body(j, _):
            onesv[j, :] = jnp.ones((16,), jnp.float32)
            return 0
        lax.fori_loop(0, CHUNK, ones_body, 0)

    plsc.subcore_barrier()

    def chunk_body(k, _):
        idxrow = srcv.at[k]
        dstrow = dstv.at[k]
        pltpu.sync_copy(xcat.at[idxrow], rows)
        pltpu.sync_copy(rows, accg.at[dstrow], add=True)
        k16 = jnp.full((16,), k, jnp.int32)

        def edge_body(j, _):
            uj = plsc.load_gather(uv, [k16, jnp.full((16,), j, jnp.int32)])
            for q in range(4):
                sl = pl.ds(q * 16, 16)
                hrows[j, sl] = rows[j, sl] * uj
            return 0
        lax.fori_loop(0, CHUNK, edge_body, 0)
        pltpu.sync_copy(hrows, acch.at[dstrow], add=True)
        if with_cnt:
            @pl.when(c == 0)
            def _():
                pltpu.sync_copy(onesv, acccnt.at[dstrow], add=True)
        return 0

    lax.fori_loop(0, NCHUNKS, chunk_body, 0)

    plsc.subcore_barrier()

    r0 = s * OROWS
    pltpu.sync_copy(accg.at[pl.ds(r0, OROWS)], g_out.at[pl.ds(c * N + r0, OROWS)])
    pltpu.sync_copy(acch.at[pl.ds(r0, OROWS)], h_out.at[pl.ds(c * N + r0, OROWS)])
    if with_cnt:
        @pl.when(c == 0)
        def _():
            pltpu.sync_copy(acccnt.at[pl.ds(r0, OROWS)], cnt_out.at[pl.ds(r0, OROWS)])


def _make_sc(with_cnt):
    outs = [
        jax.ShapeDtypeStruct((2 * N, HALF), jnp.float32),
        jax.ShapeDtypeStruct((2 * N, HALF), jnp.float32),
    ]
    if with_cnt:
        outs.append(jax.ShapeDtypeStruct((N, 16), jnp.float32))
    scratch = [
        pltpu.VMEM_SHARED((NPAD, HALF), jnp.float32),
        pltpu.VMEM_SHARED((NPAD, HALF), jnp.float32),
        pltpu.VMEM_SHARED((NPAD, 16), jnp.float32),
        pltpu.VMEM((NCHUNKS, CHUNK), jnp.int32),
        pltpu.VMEM((NCHUNKS, CHUNK), jnp.int32),
        pltpu.VMEM((NCHUNKS, CHUNK), jnp.float32),
        pltpu.VMEM((CHUNK, HALF), jnp.float32),
        pltpu.VMEM((CHUNK, HALF), jnp.float32),
        pltpu.VMEM((CHUNK, 16), jnp.float32),
    ]
    mesh = plsc.VectorSubcoreMesh(core_axis_name="c", subcore_axis_name="s")
    return pl.kernel(
        functools.partial(_sc_body, with_cnt),
        out_type=tuple(outs),
        mesh=mesh,
        scratch_types=scratch,
    )


def _dense_body(mode, hs, g, h, cnt, W, root, bias, o):
    W0 = W[0]
    W1 = W[1]
    Wd = W1 - W0
    f32 = jnp.float32
    pre = (
        jnp.dot(g[0], W0[:HALF], preferred_element_type=f32)
        + jnp.dot(g[1], W0[HALF:], preferred_element_type=f32)
        + jnp.dot(h[0], Wd[:HALF], preferred_element_type=f32)
        + jnp.dot(h[1], Wd[HALF:], preferred_element_type=f32)
    )
    c = jnp.maximum(cnt[:, 0:1], 1.0)
    r = (
        pre / c
        + jnp.dot(hs[0], root[:HALF], preferred_element_type=f32)
        + jnp.dot(hs[1], root[HALF:], preferred_element_type=f32)
        + bias[...]
    )
    if mode < 2:
        r = r * jax.nn.sigmoid(r)
        o[0] = r[:, :HALF]
        o[1] = r[:, HALF:]
    else:
        m = jnp.max(r, axis=1, keepdims=True)
        e = jnp.exp(r - m)
        sm = jnp.sum(e, axis=1, keepdims=True)
        o[...] = r - m - jnp.log(sm)


def _dense(hs, g, h, cnt, W, root, bias, mode):
    Do = W.shape[2]
    R = 1000
    nb = N // R
    if mode < 2:
        out_shape = jax.ShapeDtypeStruct((2, N, HALF), jnp.float32)
        out_spec = pl.BlockSpec((2, R, HALF), lambda i: (0, i, 0))
    else:
        out_shape = jax.ShapeDtypeStruct((N, Do), jnp.float32)
        out_spec = pl.BlockSpec((R, Do), lambda i: (i, 0))
    return pl.pallas_call(
        functools.partial(_dense_body, mode),
        grid=(nb,),
        in_specs=[
            pl.BlockSpec((2, R, HALF), lambda i: (0, i, 0)),
            pl.BlockSpec((2, R, HALF), lambda i: (0, i, 0)),
            pl.BlockSpec((2, R, HALF), lambda i: (0, i, 0)),
            pl.BlockSpec((R, 16), lambda i: (i, 0)),
            pl.BlockSpec((2, D, Do), lambda i: (0, 0, 0)),
            pl.BlockSpec((D, Do), lambda i: (0, 0)),
            pl.BlockSpec((1, Do), lambda i: (0, 0)),
        ],
        out_specs=out_spec,
        out_shape=out_shape,
    )(hs, g, h, cnt, W, root, bias)


def kernel(x, edge_index, pseudo, W0, root0, bias0, W1, root1, bias1, W2, root2, bias2):
    src = edge_index[0]
    dst = edge_index[1]
    u = pseudo[:, 0]

    pad = EPAD - E
    srcp = jnp.concatenate([src, jnp.zeros((pad,), jnp.int32)])
    dstp = jnp.concatenate([dst, jnp.full((pad,), N, jnp.int32)])
    up = jnp.concatenate([u, jnp.zeros((pad,), jnp.float32)])
    src4 = jnp.stack([srcp, srcp + N]).reshape(2, NS, NCHUNKS, CHUNK)
    dst3 = dstp.reshape(NS, NCHUNKS, CHUNK)
    u3 = up.reshape(NS, NCHUNKS, CHUNK)
    zg = jnp.zeros((ZROWS, HALF), jnp.float32)
    zc = jnp.zeros((ZROWS, 16), jnp.float32)

    sc_cnt = _make_sc(True)
    sc = _make_sc(False)

    hs = jnp.stack([x[:, :HALF], x[:, HALF:]])
    g, h, cnt = sc_cnt(hs.reshape(2 * N, HALF), src4, dst3, u3, zg, zc)
    hs = _dense(hs, g.reshape(2, N, HALF), h.reshape(2, N, HALF), cnt,
                W0, root0, bias0.reshape(1, -1), mode=0)
    g, h = sc(hs.reshape(2 * N, HALF), src4, dst3, u3, zg, zc)
    hs = _dense(hs, g.reshape(2, N, HALF), h.reshape(2, N, HALF), cnt,
                W1, root1, bias1.reshape(1, -1), mode=1)
    g, h = sc(hs.reshape(2 * N, HALF), src4, dst3, u3, zg, zc)
    out = _dense(hs, g.reshape(2, N, HALF), h.reshape(2, N, HALF), cnt,
                 W2, root2, bias2.reshape(1, -1), mode=2)
    return out


# speedup vs baseline: 2.3916x; 2.3916x over previous
import functools

import jax
import jax.numpy as jnp
from jax import lax
from jax.experimental import pallas as pl
from jax.experimental.pallas import tpu as pltpu
from jax.experimental.pallas import tpu_sc as plsc

N = 10000
E = 320000
D = 128
NS = 16

CHUNK = 64
NSPLIT = 2
HC = CHUNK // NSPLIT
NCHUNKS = 320
PER_TILE = CHUNK * NCHUNKS
EPAD = NS * PER_TILE
NPAD = 10112
ZROWS = NPAD // NS


def _sc_body(table, src3, dst3, urep, zg, gh_out,
             acc, src00, src01, src10, src11, dst0, dst1, uv0, uv1,
             rows0, rows1, isem0, isem1, gsem0, gsem1):
    c = lax.axis_index("c")
    s = lax.axis_index("s")

    pltpu.sync_copy(zg, acc.at[pl.ds(s * ZROWS, ZROWS)])

    plsc.subcore_barrier()

    srcb = ((src00, src01), (src10, src11))
    dstb = (dst0, dst1)
    uvb = (uv0, uv1)
    rowsb = (rows0, rows1)
    isem = (isem0, isem1)
    gsem = (gsem0, gsem1)

    def start_idx(k, b):
        for p in range(NSPLIT):
            pltpu.async_copy(src3.at[s, k, pl.ds(p * HC, HC)], srcb[b][p], isem[b])
        pltpu.async_copy(dst3.at[s, k], dstb[b], isem[b])

        @pl.when(c == 1)
        def _():
            pltpu.async_copy(urep.at[s, pl.ds(k * CHUNK, CHUNK)], uvb[b], isem[b])

    def wait_idx(b):
        for p in range(NSPLIT):
            pltpu.make_async_copy(src3.at[s, 0, pl.ds(0, HC)], srcb[b][p], isem[b]).wait()
        pltpu.make_async_copy(dst3.at[s, 0], dstb[b], isem[b]).wait()

        @pl.when(c == 1)
        def _():
            pltpu.make_async_copy(urep.at[s, pl.ds(0, CHUNK)], uvb[b], isem[b]).wait()

    def start_gather(b):
        for p in range(NSPLIT):
            pltpu.async_copy(table.at[srcb[b][p]], rowsb[b].at[pl.ds(p * HC, HC)], gsem[b])

    def wait_gather(b):
        for p in range(NSPLIT):
            pltpu.make_async_copy(table.at[srcb[b][p]], rowsb[b].at[pl.ds(p * HC, HC)], gsem[b]).wait()

    def process(b):
        @pl.when(c == 1)
        def _():
            @plsc.parallel_loop(0, CHUNK, 1, unroll=4)
            def _(j):
                uj = uvb[b][j, :]
                for q in range(8):
                    sl = pl.ds(q * 16, 16)
                    rowsb[b][j, sl] = rowsb[b][j, sl] * uj

        pltpu.sync_copy(rowsb[b], acc.at[dstb[b]], add=True)

    start_idx(0, 0)
    start_idx(1, 1)
    wait_idx(0)
    start_gather(0)

    def pair_body(g, _):
        k0 = 2 * g
        wait_idx(1)
        start_gather(1)
        wait_gather(0)
        process(0)

        @pl.when(k0 + 2 < NCHUNKS)
        def _():
            start_idx(k0 + 2, 0)
            wait_idx(0)
            start_gather(0)

        wait_gather(1)
        process(1)

        @pl.when(k0 + 3 < NCHUNKS)
        def _():
            start_idx(k0 + 3, 1)
        return 0

    lax.fori_loop(0, NCHUNKS // 2, pair_body, 0)

    plsc.subcore_barrier()

    r0 = s * ZROWS
    pltpu.sync_copy(acc.at[pl.ds(r0, ZROWS)], gh_out.at[c, pl.ds(r0, ZROWS)])


def _make_sc():
    outs = jax.ShapeDtypeStruct((2, NPAD, D), jnp.float32)
    scratch = [
        pltpu.VMEM_SHARED((NPAD, D), jnp.float32),
        pltpu.VMEM((HC,), jnp.int32),
        pltpu.VMEM((HC,), jnp.int32),
        pltpu.VMEM((HC,), jnp.int32),
        pltpu.VMEM((HC,), jnp.int32),
        pltpu.VMEM((CHUNK,), jnp.int32),
        pltpu.VMEM((CHUNK,), jnp.int32),
        pltpu.VMEM((CHUNK, 16), jnp.float32),
        pltpu.VMEM((CHUNK, 16), jnp.float32),
        pltpu.VMEM((CHUNK, D), jnp.float32),
        pltpu.VMEM((CHUNK, D), jnp.float32),
        pltpu.SemaphoreType.DMA,
        pltpu.SemaphoreType.DMA,
        pltpu.SemaphoreType.DMA,
        pltpu.SemaphoreType.DMA,
    ]
    mesh = plsc.VectorSubcoreMesh(core_axis_name="c", subcore_axis_name="s")
    return pl.kernel(
        _sc_body,
        out_type=outs,
        mesh=mesh,
        scratch_types=scratch,
    )


def _cnt_body(dst3, zg, cnt_out, acccnt, dstc, onesv):
    c = lax.axis_index("c")
    s = lax.axis_index("s")

    pltpu.sync_copy(zg, acccnt.at[pl.ds(s * ZROWS, ZROWS)])

    def ones_body(j, _):
        def col_body(q, _):
            onesv[j, pl.ds(q * 16, 16)] = jnp.ones((16,), jnp.float32)
            return 0
        lax.fori_loop(0, D // 16, col_body, 0)
        return 0
    lax.fori_loop(0, CHUNK, ones_body, 0)

    plsc.subcore_barrier()

    half = NCHUNKS // 2

    def chunk_body(k, _):
        pltpu.sync_copy(dst3.at[s, c * half + k], dstc)
        pltpu.sync_copy(onesv, acccnt.at[dstc], add=True)
        return 0

    lax.fori_loop(0, half, chunk_body, 0)

    plsc.subcore_barrier()

    r0 = s * ZROWS
    pltpu.sync_copy(acccnt.at[pl.ds(r0, ZROWS)], cnt_out.at[c, pl.ds(r0, ZROWS)])


def _make_cnt():
    scratch = [
        pltpu.VMEM_SHARED((NPAD, D), jnp.float32),
        pltpu.VMEM((CHUNK,), jnp.int32),
        pltpu.VMEM((CHUNK, D), jnp.float32),
    ]
    mesh = plsc.VectorSubcoreMesh(core_axis_name="c", subcore_axis_name="s")
    return pl.kernel(
        _cnt_body,
        out_type=jax.ShapeDtypeStruct((2, NPAD, D), jnp.float32),
        mesh=mesh,
        scratch_types=scratch,
    )


def _dense_body(mode, h, g, ha, cnt, W, root, bias, o):
    W0 = W[0]
    W1 = W[1]
    Wd = W1 - W0
    f32 = jnp.float32
    pre = (
        jnp.dot(g[0], W0, preferred_element_type=f32)
        + jnp.dot(ha[0], Wd, preferred_element_type=f32)
    )
    c = jnp.maximum(cnt[0, :, 0:1] + cnt[1, :, 0:1], 1.0)
    r = pre / c + jnp.dot(h[...], root[...], preferred_element_type=f32) + bias[...]
    if mode < 2:
        r = r * jax.nn.sigmoid(r)
        o[...] = r
    else:
        m = jnp.max(r, axis=1, keepdims=True)
        e = jnp.exp(r - m)
        sm = jnp.sum(e, axis=1, keepdims=True)
        o[...] = r - m - jnp.log(sm)


def _dense(h, gh, cnt, W, root, bias, mode):
    Do = W.shape[2]
    R = 1000
    nb = N // R
    return pl.pallas_call(
        functools.partial(_dense_body, mode),
        grid=(nb,),
        in_specs=[
            pl.BlockSpec((R, D), lambda i: (i, 0)),
            pl.BlockSpec((1, R, D), lambda i: (0, i, 0)),
            pl.BlockSpec((1, R, D), lambda i: (1, i, 0)),
            pl.BlockSpec((2, R, D), lambda i: (0, i, 0)),
            pl.BlockSpec((2, D, Do), lambda i: (0, 0, 0)),
            pl.BlockSpec((D, Do), lambda i: (0, 0)),
            pl.BlockSpec((1, Do), lambda i: (0, 0)),
        ],
        out_specs=pl.BlockSpec((R, Do), lambda i: (i, 0)),
        out_shape=jax.ShapeDtypeStruct((N, Do), jnp.float32),
    )(h, gh, gh, cnt, W, root, bias)


def kernel(x, edge_index, pseudo, W0, root0, bias0, W1, root1, bias1, W2, root2, bias2):
    src = edge_index[0]
    dst = edge_index[1]
    u = pseudo[:, 0]

    pad = EPAD - E
    srcp = jnp.concatenate([src, jnp.zeros((pad,), jnp.int32)])
    dstp = jnp.concatenate([dst, jnp.full((pad,), N, jnp.int32)])
    up = jnp.concatenate([u, jnp.zeros((pad,), jnp.float32)])
    src3 = srcp.reshape(NS, NCHUNKS, CHUNK)
    dst3 = dstp.reshape(NS, NCHUNKS, CHUNK)
    urep = jnp.broadcast_to(up[:, None], (EPAD, 16)).reshape(NS, NCHUNKS * CHUNK, 16)
    zg = jnp.zeros((ZROWS, D), jnp.float32)

    sc = _make_sc()
    cnt = _make_cnt()(dst3, zg)

    gh = sc(x, src3, dst3, urep, zg)
    h = _dense(x, gh, cnt, W0, root0, bias0.reshape(1, -1), mode=0)
    gh = sc(h, src3, dst3, urep, zg)
    h = _dense(h, gh, cnt, W1, root1, bias1.reshape(1, -1), mode=1)
    gh = sc(h, src3, dst3, urep, zg)
    return _dense(h, gh, cnt, W2, root2, bias2.reshape(1, -1), mode=2)
